# baseline (speedup 1.0000x reference)
import functools

import jax
import jax.numpy as jnp
from jax import lax
from jax.experimental import pallas as pl
from jax.experimental.pallas import tpu as pltpu
from jax.experimental.pallas import tpu_sc as plsc

B = 16384
D = 32
_WIDE = 128
_PACK = _WIDE // D
_NC = 2
_NS = 16
_NW = _NC * _NS
_BPW = B // _NW
_CHUNK = 128
_NCHUNK = _BPW // _CHUNK


def _gather_body(ids0, ids1, ids2, e0, e1, e2, o0, o1, o2,
                 idx0, idx1, idx2, buf0, buf1, buf2,
                 si0, si1, si2, so0, so1, so2):
    w = lax.axis_index("s") * _NC + lax.axis_index("c")
    ids = (ids0, ids1, ids2)
    tables = (e0, e1, e2)
    outs = (o0, o1, o2)
    idx = (idx0, idx1, idx2)
    buf = (buf0, buf1, buf2)
    sin = (si0, si1, si2)
    sout = (so0, so1, so2)
    for t in range(3):
        pltpu.sync_copy(ids[t].at[w], idx[t])
    ins = {}
    wr = {}
    for j in range(_NCHUNK):
        for t in range(3):
            if j >= 2:
                wr[(j - 2, t)].wait()
            ins[(j, t)] = pltpu.async_copy(
                tables[t].at[idx[t].at[j]], buf[t].at[j % 2], sin[t])
        if j >= 1:
            for t in range(3):
                ins[(j - 1, t)].wait()
                wr[(j - 1, t)] = pltpu.async_copy(
                    buf[t].at[(j - 1) % 2], outs[t].at[w, j - 1], sout[t])
    last = _NCHUNK - 1
    for t in range(3):
        ins[(last, t)].wait()
        wr[(last, t)] = pltpu.async_copy(
            buf[t].at[last % 2], outs[t].at[w, last], sout[t])
    for t in range(3):
        wr[(last - 1, t)].wait()
        wr[(last, t)].wait()


_wide_t = jax.ShapeDtypeStruct((_NW, _NCHUNK, _CHUNK, _WIDE), jnp.float32)

_gather_call = functools.partial(
    pl.kernel,
    mesh=plsc.VectorSubcoreMesh(core_axis_name="c", subcore_axis_name="s"),
    out_type=(_wide_t, _wide_t, _wide_t),
    scratch_types=[
        pltpu.VMEM((_NCHUNK, _CHUNK), jnp.int32),
        pltpu.VMEM((_NCHUNK, _CHUNK), jnp.int32),
        pltpu.VMEM((_NCHUNK, _CHUNK), jnp.int32),
        pltpu.VMEM((2, _CHUNK, _WIDE), jnp.float32),
        pltpu.VMEM((2, _CHUNK, _WIDE), jnp.float32),
        pltpu.VMEM((2, _CHUNK, _WIDE), jnp.float32),
        pltpu.SemaphoreType.DMA,
        pltpu.SemaphoreType.DMA,
        pltpu.SemaphoreType.DMA,
        pltpu.SemaphoreType.DMA,
        pltpu.SemaphoreType.DMA,
        pltpu.SemaphoreType.DMA,
    ],
)(_gather_body)


def _proj_body(w0_ref, w1_ref, w2_ref, m0_ref, m1_ref, m2_ref,
               istk_ref, w10s_ref, w21s_ref, w12s_ref, bias_ref,
               o0_ref, o1_ref, o2_ref):
    blk = w0_ref.shape[0]
    lane_grp = lax.broadcasted_iota(jnp.int32, (blk, _WIDE), 1) // D
    wm0 = w0_ref[...] * (lane_grp == m0_ref[...]).astype(jnp.float32)
    wm1 = w1_ref[...] * (lane_grp == m1_ref[...]).astype(jnp.float32)
    wm2 = w2_ref[...] * (lane_grp == m2_ref[...]).astype(jnp.float32)
    istk = istk_ref[...]
    base0 = jnp.dot(wm0, istk, preferred_element_type=jnp.float32)
    base1 = jnp.dot(wm1, istk, preferred_element_type=jnp.float32)
    base2 = jnp.dot(wm2, istk, preferred_element_type=jnp.float32)
    o0_ref[...] = base0 + jnp.dot(wm1, w10s_ref[...],
                                  preferred_element_type=jnp.float32) + bias_ref[0, :]
    o1_ref[...] = base1 + jnp.dot(wm2, w21s_ref[...],
                                  preferred_element_type=jnp.float32) + bias_ref[1, :]
    o2_ref[...] = base2 + jnp.dot(wm1, w12s_ref[...],
                                  preferred_element_type=jnp.float32) + bias_ref[2, :]


_BLK = 2048
_out_t = jax.ShapeDtypeStruct((B, D), jnp.float32)

_proj_call = pl.pallas_call(
    _proj_body,
    grid=(B // _BLK,),
    in_specs=[
        pl.BlockSpec((_BLK, _WIDE), lambda i: (i, 0)),
        pl.BlockSpec((_BLK, _WIDE), lambda i: (i, 0)),
        pl.BlockSpec((_BLK, _WIDE), lambda i: (i, 0)),
        pl.BlockSpec((_BLK, 1), lambda i: (i, 0)),
        pl.BlockSpec((_BLK, 1), lambda i: (i, 0)),
        pl.BlockSpec((_BLK, 1), lambda i: (i, 0)),
        pl.BlockSpec((_WIDE, D), lambda i: (0, 0)),
        pl.BlockSpec((_WIDE, D), lambda i: (0, 0)),
        pl.BlockSpec((_WIDE, D), lambda i: (0, 0)),
        pl.BlockSpec((_WIDE, D), lambda i: (0, 0)),
        pl.BlockSpec((8, D), lambda i: (0, 0)),
    ],
    out_specs=[
        pl.BlockSpec((_BLK, D), lambda i: (i, 0)),
        pl.BlockSpec((_BLK, D), lambda i: (i, 0)),
        pl.BlockSpec((_BLK, D), lambda i: (i, 0)),
    ],
    out_shape=(_out_t, _out_t, _out_t),
)


def kernel(level_ids_0, level_ids_1, level_ids_2, emb0, emb1, emb2,
           W01, b01, W10, b10, W12, b12, W21, b21):
    del W01, b01
    ids = [level_ids_0.astype(jnp.int32), level_ids_1.astype(jnp.int32),
           level_ids_2.astype(jnp.int32)]
    hi = [(i // _PACK).reshape(_NW, _NCHUNK, _CHUNK) for i in ids]
    lo = [(i % _PACK).reshape(B, 1) for i in ids]
    e0 = emb0.reshape(emb0.shape[0] // _PACK, _WIDE)
    e1 = emb1.reshape(emb1.shape[0] // _PACK, _WIDE)
    e2 = emb2.reshape(emb2.shape[0] // _PACK, _WIDE)
    wide0, wide1, wide2 = _gather_call(hi[0], hi[1], hi[2], e0, e1, e2)
    wide0 = wide0.reshape(B, _WIDE)
    wide1 = wide1.reshape(B, _WIDE)
    wide2 = wide2.reshape(B, _WIDE)
    eye = jnp.eye(D, dtype=jnp.float32)
    istk = jnp.tile(eye, (_PACK, 1))
    w10s = jnp.tile(W10.T, (_PACK, 1))
    w21s = jnp.tile(W21.T, (_PACK, 1))
    w12s = jnp.tile(W12.T, (_PACK, 1))
    bias = jnp.zeros((8, D), jnp.float32)
    bias = bias.at[0].set(b10).at[1].set(b21).at[2].set(b12)
    enh0, enh1, enh2 = _proj_call(wide0, wide1, wide2, lo[0], lo[1], lo[2],
                                  istk, w10s, w21s, w12s, bias)
    return (enh0, enh1, enh2)


# speedup vs baseline: 2.9043x; 2.9043x over previous
import functools

import jax
import jax.numpy as jnp
from jax import lax
from jax.experimental import pallas as pl
from jax.experimental.pallas import tpu as pltpu
from jax.experimental.pallas import tpu_sc as plsc

B = 16384
D = 32
L0, L1, L2 = 1000, 100000, 1000000
_NC = 2
_NS = 16
_NW = _NC * _NS
_V = 16
_NVEC = B // _V
_CH = 32768
_TAIL = L2 % 128
_L2A = L2 - _TAIL
_NCHUNK = -(-_L2A // _CH)
_Q = 4096
_NQ = B // _Q


def _gather_body(ids0, ids1, ids2, e0, e1, e2, o0, o1, o2, s_in, s_out):
    f = lax.axis_index("s") * _NC + lax.axis_index("c")

    def phase_emb2(idsb, dstb, tbl_a, tbl_b):
        pltpu.sync_copy(ids2, idsb)
        halves = (tbl_a, tbl_b)
        sizes = [min(_CH, _L2A - c * _CH) for c in range(_NCHUNK)]
        cps = [None, None]
        cps[0] = pltpu.async_copy(
            e2.at[f, pl.ds(0, sizes[0])], halves[0].at[pl.ds(0, sizes[0])],
            s_in)
        for c in range(_NCHUNK):
            if c + 1 < _NCHUNK:
                n = sizes[c + 1]
                cps[(c + 1) % 2] = pltpu.async_copy(
                    e2.at[f, pl.ds((c + 1) * _CH, n)],
                    halves[(c + 1) % 2].at[pl.ds(0, n)], s_in)
            cps[c % 2].wait()
            half = halves[c % 2]

            def scan(v, _, half=half, c=c):
                ids = idsb[pl.ds(v * _V, _V)]
                m = lax.shift_right_logical(ids, 15) == c
                loc = jnp.bitwise_and(ids, _CH - 1)
                g = plsc.load_gather(half, [loc], mask=m)
                pos = lax.iota(jnp.int32, _V) + v * _V
                plsc.store_scatter(dstb, [pos], g, mask=m)
                return 0

            lax.fori_loop(0, _NVEC, scan, 0)
        pltpu.sync_copy(dstb, o2.at[f])

    pl.run_scoped(
        phase_emb2,
        pltpu.VMEM((B,), jnp.int32),
        pltpu.VMEM((B,), jnp.float32),
        pltpu.VMEM((_CH,), jnp.float32),
        pltpu.VMEM((_CH,), jnp.float32),
    )

    def phase_emb01(row0, row1, idsb, dstb):
        pltpu.sync_copy(e0.at[f], row0)
        pltpu.sync_copy(e1.at[f], row1)
        for ids_hbm, row, out in ((ids0, row0, o0), (ids1, row1, o1)):
            for q in range(_NQ):
                pltpu.sync_copy(ids_hbm.at[pl.ds(q * _Q, _Q)], idsb)

                def lookup(v, _, row=row):
                    g = plsc.load_gather(row, [idsb[pl.ds(v * _V, _V)]])
                    dstb[pl.ds(v * _V, _V)] = g
                    return 0

                lax.fori_loop(0, _Q // _V, lookup, 0)
                pltpu.sync_copy(dstb, out.at[f, pl.ds(q * _Q, _Q)])

    pl.run_scoped(
        phase_emb01,
        pltpu.VMEM((L0,), jnp.float32),
        pltpu.VMEM((L1,), jnp.float32),
        pltpu.VMEM((_Q,), jnp.int32),
        pltpu.VMEM((_Q,), jnp.float32),
    )


_base_t = jax.ShapeDtypeStruct((D, B), jnp.float32)

_gather_call = functools.partial(
    pl.kernel,
    mesh=plsc.VectorSubcoreMesh(core_axis_name="c", subcore_axis_name="s"),
    compiler_params=pltpu.CompilerParams(needs_layout_passes=False),
    out_type=(_base_t, _base_t, _base_t),
    scratch_types=[
        pltpu.SemaphoreType.DMA,
        pltpu.SemaphoreType.DMA,
    ],
)(_gather_body)


def _proj_body(b0_ref, b1_ref, b2_ref, ids2_ref, tail_ref,
               w10_ref, w21_ref, w12_ref, bias_ref,
               o0_ref, o1_ref, o2_ref):
    b0 = b0_ref[...]
    b1 = b1_ref[...]
    blk = b2_ref.shape[1]
    ids2 = ids2_ref[0, :]
    rowid = lax.broadcasted_iota(jnp.int32, (_TAIL, blk), 0) + _L2A
    onehot = (rowid == ids2[None, :]).astype(jnp.float32)
    tail_b2 = jnp.dot(tail_ref[...], onehot,
                      preferred_element_type=jnp.float32)
    in_tail = (ids2 >= _L2A)[None, :]
    b2 = jnp.where(in_tail, tail_b2, b2_ref[...])
    o0_ref[...] = b0 + jnp.dot(w10_ref[...], b1,
                               preferred_element_type=jnp.float32) + bias_ref[:, 0:1]
    o1_ref[...] = b1 + jnp.dot(w21_ref[...], b2,
                               preferred_element_type=jnp.float32) + bias_ref[:, 1:2]
    o2_ref[...] = b2 + jnp.dot(w12_ref[...], b1,
                               preferred_element_type=jnp.float32) + bias_ref[:, 2:3]


_BLK = 2048
_outT_t = jax.ShapeDtypeStruct((D, B), jnp.float32)

_proj_call = pl.pallas_call(
    _proj_body,
    grid=(B // _BLK,),
    in_specs=[
        pl.BlockSpec((D, _BLK), lambda i: (0, i)),
        pl.BlockSpec((D, _BLK), lambda i: (0, i)),
        pl.BlockSpec((D, _BLK), lambda i: (0, i)),
        pl.BlockSpec((1, _BLK), lambda i: (0, i)),
        pl.BlockSpec((D, _TAIL), lambda i: (0, 0)),
        pl.BlockSpec((D, D), lambda i: (0, 0)),
        pl.BlockSpec((D, D), lambda i: (0, 0)),
        pl.BlockSpec((D, D), lambda i: (0, 0)),
        pl.BlockSpec((D, 8), lambda i: (0, 0)),
    ],
    out_specs=[
        pl.BlockSpec((D, _BLK), lambda i: (0, i)),
        pl.BlockSpec((D, _BLK), lambda i: (0, i)),
        pl.BlockSpec((D, _BLK), lambda i: (0, i)),
    ],
    out_shape=(_outT_t, _outT_t, _outT_t),
)


def kernel(level_ids_0, level_ids_1, level_ids_2, emb0, emb1, emb2,
           W01, b01, W10, b10, W12, b12, W21, b21):
    del W01, b01
    ids0 = level_ids_0.astype(jnp.int32)
    ids1 = level_ids_1.astype(jnp.int32)
    ids2 = level_ids_2.astype(jnp.int32)
    b0t, b1t, b2t = _gather_call(ids0, ids1, ids2, emb0.T, emb1.T, emb2.T)
    tail = emb2.T[:, _L2A:]
    bias = jnp.zeros((D, 8), jnp.float32)
    bias = bias.at[:, 0].set(b10).at[:, 1].set(b21).at[:, 2].set(b12)
    e0t, e1t, e2t = _proj_call(b0t, b1t, b2t, ids2.reshape(1, B), tail,
                               W10, W21, W12, bias)
    return (e0t.T, e1t.T, e2t.T)


# speedup vs baseline: 5.5904x; 1.9248x over previous
import functools

import jax
import jax.numpy as jnp
from jax import lax
from jax.experimental import pallas as pl
from jax.experimental.pallas import tpu as pltpu
from jax.experimental.pallas import tpu_sc as plsc

B = 16384
D = 32
L0, L1, L2 = 1000, 100000, 1000000
_NC = 2
_NS = 16
_NW = _NC * _NS
_V = 16
_NVEC = B // _V
_CH = 32768
_TAIL = L2 % 128
_L2A = L2 - _TAIL
_NCHUNK = -(-_L2A // _CH)
_Q = 4096
_NQ = B // _Q


def _gather_body(ids0, ids1, ids2, e0, e1, e2, o0, o1, o2, s_in, s_out):
    f = lax.axis_index("s") * _NC + lax.axis_index("c")

    def phase_emb2(idsb, dstb, tbl_a, tbl_b):
        pltpu.sync_copy(ids2, idsb)
        halves = (tbl_a, tbl_b)
        sizes = [min(_CH, _L2A - c * _CH) for c in range(_NCHUNK)]
        cps = [None, None]
        cps[0] = pltpu.async_copy(
            e2.at[f, pl.ds(0, sizes[0])], halves[0].at[pl.ds(0, sizes[0])],
            s_in)
        for c in range(_NCHUNK):
            if c + 1 < _NCHUNK:
                n = sizes[c + 1]
                cps[(c + 1) % 2] = pltpu.async_copy(
                    e2.at[f, pl.ds((c + 1) * _CH, n)],
                    halves[(c + 1) % 2].at[pl.ds(0, n)], s_in)
            cps[c % 2].wait()
            half = halves[c % 2]

            def scan(v, _, half=half, c=c):
                ids = idsb[pl.ds(v * _V, _V)]
                m = lax.shift_right_logical(ids, 15) == c
                loc = jnp.bitwise_and(ids, _CH - 1)
                g = plsc.load_gather(half, [loc], mask=m)
                pos = lax.iota(jnp.int32, _V) + v * _V
                plsc.store_scatter(dstb, [pos], g, mask=m)
                return 0

            lax.fori_loop(0, _NVEC, scan, 0, unroll=8)
        pltpu.sync_copy(dstb, o2.at[f])

    pl.run_scoped(
        phase_emb2,
        pltpu.VMEM((B,), jnp.int32),
        pltpu.VMEM((B,), jnp.float32),
        pltpu.VMEM((_CH,), jnp.float32),
        pltpu.VMEM((_CH,), jnp.float32),
    )

    def phase_emb01(row0, row1, idsb, dstb):
        pltpu.sync_copy(e0.at[f], row0)
        pltpu.sync_copy(e1.at[f], row1)
        for ids_hbm, row, out in ((ids0, row0, o0), (ids1, row1, o1)):
            for q in range(_NQ):
                pltpu.sync_copy(ids_hbm.at[pl.ds(q * _Q, _Q)], idsb)

                def lookup(v, _, row=row):
                    g = plsc.load_gather(row, [idsb[pl.ds(v * _V, _V)]])
                    dstb[pl.ds(v * _V, _V)] = g
                    return 0

                lax.fori_loop(0, _Q // _V, lookup, 0, unroll=8)
                pltpu.sync_copy(dstb, out.at[f, pl.ds(q * _Q, _Q)])

    pl.run_scoped(
        phase_emb01,
        pltpu.VMEM((L0,), jnp.float32),
        pltpu.VMEM((L1,), jnp.float32),
        pltpu.VMEM((_Q,), jnp.int32),
        pltpu.VMEM((_Q,), jnp.float32),
    )


_base_t = jax.ShapeDtypeStruct((D, B), jnp.float32)

_gather_call = functools.partial(
    pl.kernel,
    mesh=plsc.VectorSubcoreMesh(core_axis_name="c", subcore_axis_name="s"),
    compiler_params=pltpu.CompilerParams(needs_layout_passes=False),
    out_type=(_base_t, _base_t, _base_t),
    scratch_types=[
        pltpu.SemaphoreType.DMA,
        pltpu.SemaphoreType.DMA,
    ],
)(_gather_body)


def _proj_body(b0_ref, b1_ref, b2_ref, ids2_ref, tail_ref,
               w10_ref, w21_ref, w12_ref, bias_ref,
               o0_ref, o1_ref, o2_ref):
    b0 = b0_ref[...]
    b1 = b1_ref[...]
    blk = b2_ref.shape[1]
    ids2 = ids2_ref[0, :]
    rowid = lax.broadcasted_iota(jnp.int32, (_TAIL, blk), 0) + _L2A
    onehot = (rowid == ids2[None, :]).astype(jnp.float32)
    tail_b2 = jnp.dot(tail_ref[...], onehot,
                      preferred_element_type=jnp.float32)
    in_tail = (ids2 >= _L2A)[None, :]
    b2 = jnp.where(in_tail, tail_b2, b2_ref[...])
    o0_ref[...] = b0 + jnp.dot(w10_ref[...], b1,
                               preferred_element_type=jnp.float32) + bias_ref[:, 0:1]
    o1_ref[...] = b1 + jnp.dot(w21_ref[...], b2,
                               preferred_element_type=jnp.float32) + bias_ref[:, 1:2]
    o2_ref[...] = b2 + jnp.dot(w12_ref[...], b1,
                               preferred_element_type=jnp.float32) + bias_ref[:, 2:3]


_BLK = 2048
_outT_t = jax.ShapeDtypeStruct((D, B), jnp.float32)

_proj_call = pl.pallas_call(
    _proj_body,
    grid=(B // _BLK,),
    in_specs=[
        pl.BlockSpec((D, _BLK), lambda i: (0, i)),
        pl.BlockSpec((D, _BLK), lambda i: (0, i)),
        pl.BlockSpec((D, _BLK), lambda i: (0, i)),
        pl.BlockSpec((1, _BLK), lambda i: (0, i)),
        pl.BlockSpec((D, _TAIL), lambda i: (0, 0)),
        pl.BlockSpec((D, D), lambda i: (0, 0)),
        pl.BlockSpec((D, D), lambda i: (0, 0)),
        pl.BlockSpec((D, D), lambda i: (0, 0)),
        pl.BlockSpec((D, 8), lambda i: (0, 0)),
    ],
    out_specs=[
        pl.BlockSpec((D, _BLK), lambda i: (0, i)),
        pl.BlockSpec((D, _BLK), lambda i: (0, i)),
        pl.BlockSpec((D, _BLK), lambda i: (0, i)),
    ],
    out_shape=(_outT_t, _outT_t, _outT_t),
)


def kernel(level_ids_0, level_ids_1, level_ids_2, emb0, emb1, emb2,
           W01, b01, W10, b10, W12, b12, W21, b21):
    del W01, b01
    ids0 = level_ids_0.astype(jnp.int32)
    ids1 = level_ids_1.astype(jnp.int32)
    ids2 = level_ids_2.astype(jnp.int32)
    b0t, b1t, b2t = _gather_call(ids0, ids1, ids2, emb0.T, emb1.T, emb2.T)
    tail = emb2.T[:, _L2A:]
    bias = jnp.zeros((D, 8), jnp.float32)
    bias = bias.at[:, 0].set(b10).at[:, 1].set(b21).at[:, 2].set(b12)
    e0t, e1t, e2t = _proj_call(b0t, b1t, b2t, ids2.reshape(1, B), tail,
                               W10, W21, W12, bias)
    return (e0t.T, e1t.T, e2t.T)
